# Initial kernel scaffold; baseline (speedup 1.0000x reference)
import jax
import jax.numpy as jnp
from jax.experimental import pallas as pl


def kernel(x):
    raise NotImplementedError("write your pallas kernel here")


# speedup vs baseline: 2.4471x; 2.4471x over previous
import functools

import jax
import jax.numpy as jnp
from jax import lax
from jax.experimental import pallas as pl
from jax.experimental.pallas import tpu as pltpu
from jax.experimental.pallas import tpu_sc as plsc

B, N, F = 16, 200, 32
NTILE = 32
NI = (B * N) // NTILE
NBUF = 4


def _tuples_body(x_hbm, out_hbm, xb, buf, sems):
  wid = lax.axis_index("s") * 2 + lax.axis_index("c")
  b = wid // 2
  i0 = (wid % 2) * NI

  pltpu.sync_copy(x_hbm.at[b], xb)

  def fill_right(j, carry):
    v0 = xb[j, 0:16]
    v1 = xb[j, 16:32]
    for k in range(NBUF):
      buf[k, j, F : F + 16] = v0
      buf[k, j, F + 16 : 2 * F] = v1
    return carry

  lax.fori_loop(0, N, fill_right, 0)

  def step(il, carry):
    s = lax.rem(il, NBUF)
    i = i0 + il

    @pl.when(il >= NBUF)
    def _():
      pltpu.make_async_copy(buf.at[s], out_hbm.at[b, i], sems.at[s]).wait()

    v0 = xb[i, 0:16]
    v1 = xb[i, 16:32]

    def fill_left(j8, c):
      jb = j8 * 8
      for u in range(8):
        buf[s, jb + u, 0:16] = v0
        buf[s, jb + u, 16:32] = v1
      return c

    lax.fori_loop(0, N // 8, fill_left, 0)
    pltpu.async_copy(buf.at[s], out_hbm.at[b, i], sems.at[s])
    return carry

  lax.fori_loop(0, NI, step, 0)

  for k in range(NBUF):
    pltpu.make_async_copy(buf.at[k], out_hbm.at[b, 0], sems.at[k]).wait()


_tuples_sc = pl.kernel(
    _tuples_body,
    out_type=jax.ShapeDtypeStruct((B, N, N, 2 * F), jnp.float32),
    mesh=plsc.VectorSubcoreMesh(
        core_axis_name="c", subcore_axis_name="s", num_cores=2, num_subcores=16
    ),
    scratch_types=[
        pltpu.VMEM((N, F), jnp.float32),
        pltpu.VMEM((NBUF, N, 2 * F), jnp.float32),
        pltpu.SemaphoreType.DMA((NBUF,)),
    ],
)


@jax.jit
def kernel(x):
  out = _tuples_sc(x)
  return out.reshape(B, N * N, 2 * F)
